# baseline (speedup 1.0000x reference)
import jax
import jax.numpy as jnp
from jax import lax
from jax.experimental import pallas as pl
from jax.experimental.pallas import tpu as pltpu
from jax.experimental.pallas import tpu_sc as plsc

NUM_FIELDS = 26
VOCAB_P1 = 100001
D = 32
BATCH = 16384
LN_EPS = 1e-5

L = 16
NW = 32
NROWS = NUM_FIELDS * D
TPW = NROWS // NW
BCHUNK = 4096
NBC = BATCH // BCHUNK
LCHUNK = 512
NCH = BATCH // LCHUNK

_mesh = lambda: plsc.VectorSubcoreMesh(core_axis_name="c", subcore_axis_name="s")
_params = lambda: pltpu.CompilerParams(
    needs_layout_passes=False, use_tc_tiling_on_sc=True)


def _rsqrt(t):
    y = plsc.bitcast(jnp.int32(0x5F3759DF) - (plsc.bitcast(t, jnp.int32) >> 1),
                     jnp.float32)
    for _ in range(3):
        y = y * (1.5 - 0.5 * t * y * y)
    return y


def _p1_body(xt_hbm, tab_hbm, cemb_hbm, out_hbm,
             idx_v, row_v, cemb_v, obuf_v, rsem, osem0, osem1):
    wid = lax.axis_index("s") * 2 + lax.axis_index("c")
    t0 = wid * TPW
    pltpu.sync_copy(cemb_hbm, cemb_v)
    osems = (osem0, osem1)

    pltpu.async_copy(tab_hbm.at[t0], row_v, rsem)
    pltpu.sync_copy(xt_hbm.at[t0 // D], idx_v)

    def wait_row():
        pltpu.make_async_copy(tab_hbm.at[t0], row_v, rsem).wait()

    def wait_out(p):
        pltpu.make_async_copy(
            obuf_v.at[p], out_hbm.at[t0, pl.ds(0, BCHUNK)], osems[p]).wait()

    def task(k, carry):
        t = t0 + k
        cvec = plsc.load_gather(cemb_v, [jnp.zeros((L,), jnp.int32) + t])
        wait_row()

        for c in range(NBC):
            p = c % 2
            if c < 2:
                @pl.when(k > 0)
                def _():
                    wait_out(p)
            else:
                wait_out(p)

            @plsc.parallel_loop(0, BCHUNK, L, unroll=16)
            def _gather(i, _c=c, _p=p, _cvec=cvec):
                o = pl.multiple_of(i, L)
                iv = idx_v[pl.ds(_c * BCHUNK + o, L)] + 1
                v = plsc.load_gather(row_v, [iv]) + _cvec
                obuf_v[_p, pl.ds(o, L)] = v

            pltpu.async_copy(
                obuf_v.at[p], out_hbm.at[t, pl.ds(c * BCHUNK, BCHUNK)],
                osems[p])

        @pl.when(k + 1 < TPW)
        def _start_next():
            tn = t + 1
            pltpu.async_copy(tab_hbm.at[tn], row_v, rsem)

            @pl.when(tn // D != t // D)
            def _reload_x():
                pltpu.sync_copy(xt_hbm.at[tn // D], idx_v)

        return carry

    lax.fori_loop(0, TPW, task, 0)
    wait_out(0)
    wait_out(1)


def _p2_body(pre_hbm, w_hbm, b_hbm, out_hbm,
             ibuf_v, obuf_v, w_v, b_v, isem0, isem1, osem0, osem1):
    wid = lax.axis_index("s") * 2 + lax.axis_index("c")
    t0 = wid * TPW
    pltpu.sync_copy(w_hbm, w_v)
    pltpu.sync_copy(b_hbm, b_v)
    w_lo = w_v[pl.ds(0, L)]
    w_hi = w_v[pl.ds(L, L)]
    b_lo = b_v[pl.ds(0, L)]
    b_hi = b_v[pl.ds(L, L)]
    w_s = [w_lo[d] for d in range(L)] + [w_hi[d] for d in range(L)]
    b_s = [b_lo[d] for d in range(L)] + [b_hi[d] for d in range(L)]

    isems = (isem0, isem1)
    osems = (osem0, osem1)

    def slices(t):
        f = t // NCH
        co = pl.multiple_of((t % NCH) * LCHUNK, LCHUNK)
        return (pre_hbm.at[f, :, pl.ds(co, LCHUNK)],
                out_hbm.at[f, :, pl.ds(co, LCHUNK)])

    def start_in(t, p):
        src, _ = slices(t)
        pltpu.async_copy(src, ibuf_v.at[p], isems[p])

    def wait_in(p):
        src, _ = slices(t0)
        pltpu.make_async_copy(src, ibuf_v.at[p], isems[p]).wait()

    def wait_out(p):
        _, dst = slices(t0)
        pltpu.make_async_copy(obuf_v.at[p], dst, osems[p]).wait()

    start_in(t0, 0)
    start_in(t0 + 1, 1)

    def pair(kk, carry):
        for j in range(2):
            _task(kk * 2 + j, j)
        return carry

    def _task(k, p):
        t = t0 + k
        wait_in(p)

        @pl.when(k >= 2)
        def _drain_out():
            wait_out(p)

        @plsc.parallel_loop(0, LCHUNK, L, unroll=8)
        def _ln(g, _p=p):
            o = pl.multiple_of(g, L)
            s = None
            s2 = None
            for d in range(D):
                v = ibuf_v[_p, d, pl.ds(o, L)]
                s = v if d == 0 else s + v
                s2 = v * v if d == 0 else s2 + v * v
            mean = s * (1.0 / D)
            var = s2 * (1.0 / D) - mean * mean
            rstd = _rsqrt(var + LN_EPS)
            for d in range(D):
                v = ibuf_v[_p, d, pl.ds(o, L)]
                obuf_v[_p, d, pl.ds(o, L)] = (v - mean) * (rstd * w_s[d]) + b_s[d]

        _, dst = slices(t)
        pltpu.async_copy(obuf_v.at[p], dst, osems[p])

        @pl.when(k + 2 < TPW)
        def _prefetch():
            start_in(t + 2, p)

    lax.fori_loop(0, TPW // 2, pair, 0)
    wait_out(0)
    wait_out(1)


@jax.jit
def _run(xt, tab2, cembf, ln_w, ln_b):
    pre = pl.kernel(
        _p1_body,
        out_type=jax.ShapeDtypeStruct((NROWS, BATCH), jnp.float32),
        mesh=_mesh(),
        scratch_types=[
            pltpu.VMEM((BATCH,), jnp.int32),
            pltpu.VMEM((VOCAB_P1,), jnp.float32),
            pltpu.VMEM((NROWS,), jnp.float32),
            pltpu.VMEM((2, BCHUNK), jnp.float32),
            pltpu.SemaphoreType.DMA,
            pltpu.SemaphoreType.DMA,
            pltpu.SemaphoreType.DMA,
        ],
        compiler_params=_params(),
    )(xt, tab2, cembf)
    out = pl.kernel(
        _p2_body,
        out_type=jax.ShapeDtypeStruct((NUM_FIELDS, D, BATCH), jnp.float32),
        mesh=_mesh(),
        scratch_types=[
            pltpu.VMEM((2, D, LCHUNK), jnp.float32),
            pltpu.VMEM((2, D, LCHUNK), jnp.float32),
            pltpu.VMEM((D,), jnp.float32),
            pltpu.VMEM((D,), jnp.float32),
            pltpu.SemaphoreType.DMA,
            pltpu.SemaphoreType.DMA,
            pltpu.SemaphoreType.DMA,
            pltpu.SemaphoreType.DMA,
        ],
        compiler_params=_params(),
    )(pre.reshape(NUM_FIELDS, D, BATCH), ln_w, ln_b)
    return out


def kernel(x, tables, column_embedding, ln_weight, ln_bias):
    xt = x.astype(jnp.int32).T
    tab2 = tables.transpose(0, 2, 1).reshape(NROWS, VOCAB_P1)
    cembf = column_embedding.reshape(NROWS)
    out = _run(xt, tab2, cembf, ln_weight, ln_bias)
    return out.transpose(2, 0, 1)


# speedup vs baseline: 1.0991x; 1.0991x over previous
import jax
import jax.numpy as jnp
from jax import lax
from jax.experimental import pallas as pl
from jax.experimental.pallas import tpu as pltpu
from jax.experimental.pallas import tpu_sc as plsc

NUM_FIELDS = 26
VOCAB_P1 = 100001
D = 32
BATCH = 16384
LN_EPS = 1e-5

L = 16
NW = 32
NROWS = NUM_FIELDS * D
TPW = NROWS // NW
BCHUNK = 4096
NBC = BATCH // BCHUNK
LCHUNK = 512
NCH = BATCH // LCHUNK

_mesh = lambda: plsc.VectorSubcoreMesh(core_axis_name="c", subcore_axis_name="s")
_params = lambda: pltpu.CompilerParams(
    needs_layout_passes=False, use_tc_tiling_on_sc=True)


def _rsqrt(t):
    y = plsc.bitcast(jnp.int32(0x5F3759DF) - (plsc.bitcast(t, jnp.int32) >> 1),
                     jnp.float32)
    for _ in range(3):
        y = y * (1.5 - 0.5 * t * y * y)
    return y


def _body(xt_hbm, tab_hbm, cemb_hbm, w_hbm, b_hbm, out_hbm,
          rsem, sem0, sem1, sem2, sem3):
    wid = lax.axis_index("c") * 16 + lax.axis_index("s")
    t0 = wid * TPW

    def p1(idx_v, row_v, cemb_v, obuf_v):
        osems = (sem0, sem1)
        pltpu.sync_copy(cemb_hbm, cemb_v)
        pltpu.async_copy(tab_hbm.at[t0], row_v, rsem)
        pltpu.sync_copy(xt_hbm.at[t0 // D], idx_v)

        def wait_row():
            pltpu.make_async_copy(tab_hbm.at[t0], row_v, rsem).wait()

        def wait_out(p):
            pltpu.make_async_copy(
                obuf_v.at[p], out_hbm.at[0, 0, pl.ds(0, BCHUNK)],
                osems[p]).wait()

        def task(k, carry):
            t = t0 + k
            f = t // D
            d = t % D
            cvec = plsc.load_gather(cemb_v, [jnp.zeros((L,), jnp.int32) + t])
            wait_row()

            for c in range(NBC):
                p = c % 2
                if c < 2:
                    @pl.when(k > 0)
                    def _():
                        wait_out(p)
                else:
                    wait_out(p)

                @plsc.parallel_loop(0, BCHUNK, L, unroll=8)
                def _gather(i, _c=c, _p=p, _cvec=cvec):
                    o = pl.multiple_of(i, L)
                    iv = idx_v[pl.ds(_c * BCHUNK + o, L)] + 1
                    v = plsc.load_gather(row_v, [iv]) + _cvec
                    obuf_v[_p, pl.ds(o, L)] = v

                pltpu.async_copy(
                    obuf_v.at[p], out_hbm.at[f, d, pl.ds(c * BCHUNK, BCHUNK)],
                    osems[p])

            @pl.when(k + 1 < TPW)
            def _start_next():
                tn = t + 1
                pltpu.async_copy(tab_hbm.at[tn], row_v, rsem)

                @pl.when(tn // D != f)
                def _reload_x():
                    pltpu.sync_copy(xt_hbm.at[tn // D], idx_v)

            return carry

        lax.fori_loop(0, TPW, task, 0)
        wait_out(0)
        wait_out(1)

    pl.run_scoped(
        p1,
        pltpu.VMEM((BATCH,), jnp.int32),
        pltpu.VMEM((VOCAB_P1,), jnp.float32),
        pltpu.VMEM((NROWS,), jnp.float32),
        pltpu.VMEM((2, BCHUNK), jnp.float32),
    )

    plsc.subcore_barrier()

    def p2(ibuf_v, obuf_v, w_v, b_v):
        isems = (sem0, sem1)
        osems = (sem2, sem3)
        pltpu.sync_copy(w_hbm, w_v)
        pltpu.sync_copy(b_hbm, b_v)
        w_lo = w_v[pl.ds(0, L)]
        w_hi = w_v[pl.ds(L, L)]
        b_lo = b_v[pl.ds(0, L)]
        b_hi = b_v[pl.ds(L, L)]
        w_s = [w_lo[d] for d in range(L)] + [w_hi[d] for d in range(L)]
        b_s = [b_lo[d] for d in range(L)] + [b_hi[d] for d in range(L)]

        def slices(t):
            f = t // NCH
            co = pl.multiple_of((t % NCH) * LCHUNK, LCHUNK)
            return out_hbm.at[f, :, pl.ds(co, LCHUNK)]

        def start_in(t, p):
            pltpu.async_copy(slices(t), ibuf_v.at[p], isems[p])

        def wait_in(p):
            pltpu.make_async_copy(slices(t0), ibuf_v.at[p], isems[p]).wait()

        def wait_out(p):
            pltpu.make_async_copy(obuf_v.at[p], slices(t0), osems[p]).wait()

        start_in(t0, 0)
        start_in(t0 + 1, 1)

        def pair(kk, carry):
            for j in range(2):
                _task(kk * 2 + j, j)
            return carry

        def _task(k, p):
            t = t0 + k
            wait_in(p)

            @pl.when(k >= 2)
            def _drain_out():
                wait_out(p)

            @plsc.parallel_loop(0, LCHUNK, L, unroll=4)
            def _ln(g, _p=p):
                o = pl.multiple_of(g, L)
                s = None
                s2 = None
                for d in range(D):
                    v = ibuf_v[_p, d, pl.ds(o, L)]
                    s = v if d == 0 else s + v
                    s2 = v * v if d == 0 else s2 + v * v
                mean = s * (1.0 / D)
                var = s2 * (1.0 / D) - mean * mean
                rstd = _rsqrt(var + LN_EPS)
                for d in range(D):
                    v = ibuf_v[_p, d, pl.ds(o, L)]
                    obuf_v[_p, d, pl.ds(o, L)] = \
                        (v - mean) * (rstd * w_s[d]) + b_s[d]

            pltpu.async_copy(obuf_v.at[p], slices(t), osems[p])

            @pl.when(k + 2 < TPW)
            def _prefetch():
                start_in(t + 2, p)

        lax.fori_loop(0, TPW // 2, pair, 0)
        wait_out(0)
        wait_out(1)

    pl.run_scoped(
        p2,
        pltpu.VMEM((2, D, LCHUNK), jnp.float32),
        pltpu.VMEM((2, D, LCHUNK), jnp.float32),
        pltpu.VMEM((D,), jnp.float32),
        pltpu.VMEM((D,), jnp.float32),
    )


@jax.jit
def _run(xt, tab2, cembf, ln_w, ln_b):
    out = pl.kernel(
        _body,
        out_type=jax.ShapeDtypeStruct((NUM_FIELDS, D, BATCH), jnp.float32),
        mesh=_mesh(),
        scratch_types=[
            pltpu.SemaphoreType.DMA,
            pltpu.SemaphoreType.DMA,
            pltpu.SemaphoreType.DMA,
            pltpu.SemaphoreType.DMA,
            pltpu.SemaphoreType.DMA,
        ],
        compiler_params=_params(),
    )(xt, tab2, cembf, ln_w, ln_b)
    return out


def kernel(x, tables, column_embedding, ln_weight, ln_bias):
    xt = x.astype(jnp.int32).T
    tab2 = tables.transpose(0, 2, 1).reshape(NROWS, VOCAB_P1)
    cembf = column_embedding.reshape(NROWS)
    out = _run(xt, tab2, cembf, ln_weight, ln_bias)
    return out.transpose(2, 0, 1)
